# Initial kernel scaffold; baseline (speedup 1.0000x reference)
import jax
import jax.numpy as jnp
from jax.experimental import pallas as pl


def kernel(x, y, ln_x_w, ln_x_b, ln_y_w, ln_y_b, w_qkv_x, w_qkv_y, w_dw_x, w_dw_y, t1, t2, t3, w_proj, norm_w, norm_b, w_ffn_in, w_ffn_dw, w_ffn_out, w_fuse1, b_fuse1, w_fuse2, b_fuse2, bn_w, bn_b):
    raise NotImplementedError("write your pallas kernel here")


# speedup vs baseline: 1.1645x; 1.1645x over previous
import functools
import numpy as np
import jax
import jax.numpy as jnp
from jax import lax
from jax.experimental import pallas as pl
from jax.experimental.pallas import tpu as pltpu


def _ln_last(t, w, b):
    mu = jnp.mean(t, axis=-1, keepdims=True)
    var = jnp.mean(jnp.square(t - mu), axis=-1, keepdims=True)
    return (t - mu) * lax.rsqrt(var + 1e-5) * w + b


def _l2n_rows(v):
    ss = jnp.sum(v * v, axis=-1, keepdims=True)
    return v * lax.rsqrt(jnp.maximum(ss, 1e-24))


def _softmax_rows(s):
    m = jnp.max(s, axis=-1, keepdims=True)
    e = jnp.exp(s - m)
    return e * pl.reciprocal(jnp.sum(e, axis=-1, keepdims=True), approx=True)


def _erf(x):
    a1, a2, a3, a4, a5 = 0.254829592, -0.284496736, 1.421413741, -1.453152027, 1.061405429
    p = 0.3275911
    s = jnp.sign(x)
    z = jnp.abs(x)
    t = pl.reciprocal(1.0 + p * z, approx=True)
    poly = t * (a1 + t * (a2 + t * (a3 + t * (a4 + t * a5))))
    return s * (1.0 - poly * jnp.exp(-z * z))


def _gelu(x):
    return 0.5 * x * (1.0 + _erf(x * 0.7071067811865476))


def _bdot(a, b):
    return jnp.dot(a.astype(jnp.bfloat16), b.astype(jnp.bfloat16),
                   preferred_element_type=jnp.float32)


def _bdot_t(a, b):
    return lax.dot_general(a.astype(jnp.bfloat16), b.astype(jnp.bfloat16),
                           dimension_numbers=(((1,), (1,)), ((), ())),
                           preferred_element_type=jnp.float32)


def _dw3x3_aligned(v, w, cb, lb, rb, mL, mR, HW, W):
    Cg = v.shape[1]
    zhalo = jnp.zeros((W, Cg), jnp.float32)
    cb[0:W, 0:Cg] = zhalo
    cb[W + HW:2 * W + HW, 0:Cg] = zhalo
    lb[0:W, 0:Cg] = zhalo
    lb[W + HW:2 * W + HW, 0:Cg] = zhalo
    rb[0:W, 0:Cg] = zhalo
    rb[W + HW:2 * W + HW, 0:Cg] = zhalo
    cb[W:W + HW, 0:Cg] = v
    lb[W:W + HW, 0:Cg] = cb[W - 1:W - 1 + HW, 0:Cg] * mL
    rb[W:W + HW, 0:Cg] = cb[W + 1:W + 1 + HW, 0:Cg] * mR
    acc = jnp.zeros((HW, Cg), jnp.float32)
    for ki in range(3):
        o = W * ki
        acc = (acc
               + lb[o:o + HW, 0:Cg] * w[3 * ki + 0][None, :]
               + cb[o:o + HW, 0:Cg] * w[3 * ki + 1][None, :]
               + rb[o:o + HW, 0:Cg] * w[3 * ki + 2][None, :])
    return acc


def _main_kernel(
        x_ref, y_ref, selT_ref,
        lnxw_ref, lnxb_ref, lnyw_ref, lnyb_ref,
        wqkvx_ref, wqkvy_ref, wdwx_ref, wdwy_ref,
        t1r_ref, t2r_ref, t3r_ref, hmask_ref,
        wproj_ref, nw_ref, nb_ref,
        wfi_ref, wfd_ref, wfo_ref,
        wf1_ref, bf1_ref, wf2_ref, bf2_ref,
        z_ref, stats_ref,
        cb, lb, rb, fc, fl, fr,
        *, H, W):
    C, HW = x_ref.shape
    C3 = wqkvx_ref.shape[1]
    hid2 = wfi_ref.shape[1]
    hid = hid2 // 2

    col = lax.broadcasted_iota(jnp.int32, (HW, 1), 0) % W
    mL = (col != 0).astype(jnp.float32)
    mR = (col != W - 1).astype(jnp.float32)

    x_cl = jnp.transpose(x_ref[...])
    y_cl = jnp.transpose(y_ref[...])

    def branch(t_cl, lnw, lnb, wqkv_ref, wdw_ref):
        tn = _ln_last(t_cl, lnw, lnb)
        qkv = _bdot(tn, wqkv_ref[...])
        dw = _dw3x3_aligned(qkv, wdw_ref[...], cb, lb, rb, mL, mR, HW, W)
        return jnp.transpose(dw)

    dx = branch(x_cl, lnxw_ref[...], lnxb_ref[...], wqkvx_ref, wdwx_ref)
    dy = branch(y_cl, lnyw_ref[...], lnyb_ref[...], wqkvy_ref, wdwy_ref)
    qx, kx, vx = dx[0:C], dx[C:2 * C], dx[2 * C:3 * C]
    qy, ky, vy = dy[0:C], dy[C:2 * C], dy[2 * C:3 * C]

    selT = selT_ref[...]
    kxs = _bdot(kx, selT)
    vxs = _bdot(vx, selT)
    qys = _bdot(qy, selT)

    hmask = hmask_ref[...]
    qxn = _l2n_rows(qx)
    kyn = _l2n_rows(ky)
    qyn = _l2n_rows(qys)
    kxn = _l2n_rows(kxs)

    s1 = _bdot_t(qxn, kyn) * t1r_ref[...]
    attnx = _softmax_rows(jnp.where(hmask > 0.5, s1, -1e30))
    s2 = _bdot_t(qyn, kxn) * t2r_ref[...]
    attny = _softmax_rows(jnp.where(hmask > 0.5, s2, -1e30))

    a2 = _bdot(attnx, attny)
    t2v = _bdot(a2, vxs)
    s3 = _bdot_t(t2v, vxs) * hmask * t3r_ref[...]
    attn = _bdot(s3, vy)

    out = x_cl + _bdot(jnp.transpose(attn), wproj_ref[...])

    on = _ln_last(out, nw_ref[...], nb_ref[...])
    pin = _bdot(on, wfi_ref[...])
    dwf = _dw3x3_aligned(pin, wfd_ref[...], cb, lb, rb, mL, mR, HW, W)
    g = _gelu(dwf[:, 0:hid]) * dwf[:, hid:hid2]
    out = out + _bdot(g, wfo_ref[...])

    z0 = x_cl + x_cl * out
    z1 = _bdot(z0, wf1_ref[...]) + bf1_ref[...]

    zb = jnp.zeros((W, C), jnp.bfloat16)
    fc[0:W, :] = zb
    fc[W + HW:2 * W + HW, :] = zb
    fl[0:W, :] = zb
    fl[W + HW:2 * W + HW, :] = zb
    fr[0:W, :] = zb
    fr[W + HW:2 * W + HW, :] = zb
    fc[W:W + HW, :] = z1.astype(jnp.bfloat16)
    fl[W:W + HW, :] = fc[W - 1:W - 1 + HW, :] * mL.astype(jnp.bfloat16)
    fr[W:W + HW, :] = fc[W + 1:W + 1 + HW, :] * mR.astype(jnp.bfloat16)
    z2 = jnp.broadcast_to(bf2_ref[...], (HW, C)).astype(jnp.float32)
    for ki in range(3):
        o = W * ki
        z2 = z2 + jnp.dot(fl[o:o + HW, :], wf2_ref[3 * ki + 0],
                          preferred_element_type=jnp.float32)
        z2 = z2 + jnp.dot(fc[o:o + HW, :], wf2_ref[3 * ki + 1],
                          preferred_element_type=jnp.float32)
        z2 = z2 + jnp.dot(fr[o:o + HW, :], wf2_ref[3 * ki + 2],
                          preferred_element_type=jnp.float32)

    z_ref[...] = jnp.transpose(z2)

    s_sum = jnp.sum(z2, axis=0, keepdims=True)
    s_sq = jnp.sum(z2 * z2, axis=0, keepdims=True)
    stats_ref[...] = jnp.concatenate(
        [s_sum, s_sq, jnp.zeros((6, C), jnp.float32)], axis=0)


def _bn_relu_kernel(z_ref, scale_ref, shift_ref, o_ref):
    o_ref[...] = jnp.maximum(z_ref[...] * scale_ref[...] + shift_ref[...], 0.0)


def kernel(x, y, ln_x_w, ln_x_b, ln_y_w, ln_y_b, w_qkv_x, w_qkv_y, w_dw_x,
           w_dw_y, t1, t2, t3, w_proj, norm_w, norm_b, w_ffn_in, w_ffn_dw,
           w_ffn_out, w_fuse1, b_fuse1, w_fuse2, b_fuse2, bn_w, bn_b):
    B, C, H, W = x.shape
    HW = H * W
    Ho, Wo = (H + 1) // 2, (W + 1) // 2
    HWs = Ho * Wo
    C3 = 3 * C
    hid = w_ffn_out.shape[0]
    hid2 = 2 * hid
    num_heads = t1.shape[0]
    hc = C // num_heads

    x2 = x.reshape(B, C, HW)
    y2 = y.reshape(B, C, HW)

    sel = np.zeros((HW, HWs), np.float32)
    pos = (2 * (np.arange(HWs) // Wo)) * W + 2 * (np.arange(HWs) % Wo)
    sel[pos, np.arange(HWs)] = 1.0
    selT = jnp.asarray(sel, jnp.bfloat16)

    hm = (np.arange(C)[:, None] // hc == np.arange(C)[None, :] // hc)
    hmask = jnp.asarray(hm.astype(np.float32))
    t1r = jnp.repeat(t1.reshape(num_heads), hc).reshape(C, 1)
    t2r = jnp.repeat(t2.reshape(num_heads), hc).reshape(C, 1)
    t3r = jnp.repeat(t3.reshape(num_heads), hc).reshape(C, 1)

    bf16 = jnp.bfloat16
    wqkvx_b = w_qkv_x.astype(bf16)
    wqkvy_b = w_qkv_y.astype(bf16)
    wproj_b = w_proj.astype(bf16)
    wfi_b = w_ffn_in.astype(bf16)
    wfo_b = w_ffn_out.astype(bf16)
    wf1_b = w_fuse1.astype(bf16)
    wf2_b = w_fuse2.astype(bf16)

    wspec = lambda *shape: pl.BlockSpec(shape, lambda b, s=shape: (0,) * len(s))
    bspec = lambda *shape: pl.BlockSpec((None,) + shape,
                                        lambda b, s=shape: (b,) + (0,) * len(s))

    kfn = functools.partial(_main_kernel, H=H, W=W)
    pad = HW + 2 * W
    wide = max(C3, hid2)

    z, stats = pl.pallas_call(
        kfn,
        out_shape=[jax.ShapeDtypeStruct((B, C, HW), jnp.float32),
                   jax.ShapeDtypeStruct((B, 8, C), jnp.float32)],
        grid=(B,),
        in_specs=[
            bspec(C, HW), bspec(C, HW), wspec(HW, HWs),
            wspec(1, C), wspec(1, C), wspec(1, C), wspec(1, C),
            wspec(C, C3), wspec(C, C3), wspec(9, C3), wspec(9, C3),
            wspec(C, 1), wspec(C, 1), wspec(C, 1), wspec(C, C),
            wspec(C, C), wspec(1, C), wspec(1, C),
            wspec(C, hid2), wspec(9, hid2), wspec(hid, C),
            wspec(C, C), wspec(1, C), wspec(9, C, C), wspec(1, C),
        ],
        out_specs=[bspec(C, HW), bspec(8, C)],
        scratch_shapes=[
            pltpu.VMEM((pad, wide), jnp.float32),
            pltpu.VMEM((pad, wide), jnp.float32),
            pltpu.VMEM((pad, wide), jnp.float32),
            pltpu.VMEM((pad, C), jnp.bfloat16),
            pltpu.VMEM((pad, C), jnp.bfloat16),
            pltpu.VMEM((pad, C), jnp.bfloat16),
        ],
        compiler_params=pltpu.CompilerParams(
            dimension_semantics=("parallel",),
            vmem_limit_bytes=32 * 1024 * 1024),
    )(x2, y2, selT,
      ln_x_w, ln_x_b, ln_y_w, ln_y_b,
      wqkvx_b, wqkvy_b, w_dw_x, w_dw_y,
      t1r, t2r, t3r, hmask,
      wproj_b, norm_w, norm_b,
      wfi_b, w_ffn_dw, wfo_b,
      wf1_b, b_fuse1, wf2_b, b_fuse2)

    n = B * HW
    mean = jnp.sum(stats[:, 0, :], axis=0) / n
    var = jnp.maximum(jnp.sum(stats[:, 1, :], axis=0) / n - mean * mean, 0.0)
    inv = lax.rsqrt(var + 1e-5)
    bw = bn_w.reshape(-1)
    bb = bn_b.reshape(-1)
    scale = (bw * inv).reshape(C, 1)
    shift = (bb - mean * bw * inv).reshape(C, 1)

    out = pl.pallas_call(
        _bn_relu_kernel,
        out_shape=jax.ShapeDtypeStruct((B, C, HW), jnp.float32),
        grid=(B,),
        in_specs=[pl.BlockSpec((None, C, HW), lambda b: (b, 0, 0)),
                  pl.BlockSpec((C, 1), lambda b: (0, 0)),
                  pl.BlockSpec((C, 1), lambda b: (0, 0))],
        out_specs=pl.BlockSpec((None, C, HW), lambda b: (b, 0, 0)),
        compiler_params=pltpu.CompilerParams(dimension_semantics=("parallel",)),
    )(z, scale, shift)

    return out.reshape(B, C, H, W)
